# baseline (speedup 1.0000x reference)
import functools

import jax
import jax.numpy as jnp
from jax import lax
from jax.experimental import pallas as pl
from jax.experimental.pallas import tpu as pltpu
from jax.experimental.pallas import tpu_sc as plsc

D_MODEL = 1024
MAX_TIME = 4096
N_KINDS = 2
NC, NS, L = 2, 16, 16
NW = NC * NS
CH = 16
NB = 4
TR = 1024


def _combine_table(time_emb, view_emb, kind_emb):
    nblk = MAX_TIME // TR

    def body(te, ve, ke, out):
        base = te[...] + ve[...]
        out[0] = base + ke[0:1, :]
        out[1] = base + ke[1:2, :]

    ct3 = pl.pallas_call(
        body,
        grid=(nblk,),
        in_specs=[
            pl.BlockSpec((TR, D_MODEL), lambda i: (i, 0)),
            pl.BlockSpec((1, D_MODEL), lambda i: (0, 0)),
            pl.BlockSpec((N_KINDS, D_MODEL), lambda i: (0, 0)),
        ],
        out_specs=pl.BlockSpec((N_KINDS, TR, D_MODEL), lambda i: (0, i, 0)),
        out_shape=jax.ShapeDtypeStruct((N_KINDS, MAX_TIME, D_MODEL),
                                       jnp.float32),
    )(time_emb, view_emb, kind_emb)
    return ct3.reshape(N_KINDS * MAX_TIME, D_MODEL)


def _gather_kernel(n_tok: int):
    tpw = n_tok // NW
    nch = tpw // CH

    mesh = plsc.VectorSubcoreMesh(core_axis_name="c", subcore_axis_name="s")

    scratch = [
        pltpu.VMEM((nch, CH), jnp.int32),
        pltpu.VMEM((nch, CH), jnp.int32),
    ]
    scratch += [pltpu.VMEM((CH, 8, D_MODEL // 8), jnp.float32)
                for _ in range(NB)]
    scratch += [pltpu.SemaphoreType.DMA for _ in range(2 * NB)]

    @functools.partial(
        pl.kernel,
        mesh=mesh,
        out_type=jax.ShapeDtypeStruct((n_tok, 8, D_MODEL // 8), jnp.float32),
        scratch_types=scratch,
    )
    def k(t_hbm, kind_hbm, ct_hbm, out_hbm, idx_v, kind_v, *bufs_and_sems):
        bufs = bufs_and_sems[:NB]
        gsem = bufs_and_sems[NB:2 * NB]
        wsem = bufs_and_sems[2 * NB:]

        wid = lax.axis_index("s") * NC + lax.axis_index("c")
        base_row = wid * nch

        pltpu.sync_copy(t_hbm.at[pl.ds(base_row, nch)], idx_v)
        pltpu.sync_copy(kind_hbm.at[pl.ds(base_row, nch)], kind_v)

        zero = jnp.zeros((L,), jnp.int32)
        def fuse_row(r, _):
            for g in range(CH // L):
                sl = pl.ds(g * L, L)
                tt = jnp.minimum(jnp.maximum(idx_v[r, sl], zero),
                                 jnp.full((L,), MAX_TIME - 1, jnp.int32))
                kk = jnp.minimum(jnp.maximum(kind_v[r, sl], zero),
                                 jnp.full((L,), N_KINDS - 1, jnp.int32))
                idx_v[r, sl] = tt + kk * MAX_TIME
            return 0
        lax.fori_loop(0, nch, fuse_row, 0, unroll=False)

        def gather(c, b):
            return pltpu.async_copy(ct_hbm.at[idx_v.at[c]], bufs[b], gsem[b])

        def wb_start(c, b):
            return pltpu.async_copy(
                bufs[b], out_hbm.at[pl.ds(wid * tpw + c * CH, CH)], wsem[b])

        def wb_wait(c, b):
            pltpu.make_async_copy(
                bufs[b], out_hbm.at[pl.ds(wid * tpw + c * CH, CH)],
                wsem[b]).wait()

        for c in range(NB - 1):
            gather(c, c)

        def process(c, b):
            pltpu.make_async_copy(
                ct_hbm.at[idx_v.at[c]], bufs[b], gsem[b]).wait()
            wb_start(c, b)

            b_next = (b + NB - 1) % NB

            @pl.when(c + NB - 1 < nch)
            def _():
                @pl.when(c >= 1)
                def _():
                    wb_wait(c - 1, b_next)
                gather(c + NB - 1, b_next)

        def outer(co, _):
            for b in range(NB):
                process(co * NB + b, b)
            return 0

        lax.fori_loop(0, nch // NB, outer, 0, unroll=False)

        for i in range(NB):
            c = nch - NB + i
            wb_wait(c, c % NB)

    return k


def kernel(t, kind_id, view_id, time_emb, view_emb, kind_emb):
    del view_id
    b, s = t.shape
    n_tok = b * s
    t2 = t.reshape(n_tok // CH, CH).astype(jnp.int32)
    k2 = kind_id.reshape(n_tok // CH, CH).astype(jnp.int32)
    ct = _combine_table(time_emb, view_emb, kind_emb)
    ct = ct.reshape(N_KINDS * MAX_TIME, 8, D_MODEL // 8)
    out = _gather_kernel(n_tok)(t2, k2, ct)
    return out.reshape(b, s, D_MODEL)


# speedup vs baseline: 2.1967x; 2.1967x over previous
import functools

import jax
import jax.numpy as jnp
from jax import lax
from jax.experimental import pallas as pl
from jax.experimental.pallas import tpu as pltpu
from jax.experimental.pallas import tpu_sc as plsc

D_MODEL = 1024
MAX_TIME = 4096
N_KINDS = 2
NC, NS, L = 2, 16, 16
NW = NC * NS
CH = 16
NB = 4
TR = 1024


def _combine_table(time_emb, view_emb, kind_emb):
    nblk = MAX_TIME // TR

    def body(te, ve, ke, out):
        base = te[...] + ve[...]
        out[0] = base + ke[0:1, :]
        out[1] = base + ke[1:2, :]

    ct3 = pl.pallas_call(
        body,
        grid=(nblk,),
        in_specs=[
            pl.BlockSpec((TR, D_MODEL), lambda i: (i, 0)),
            pl.BlockSpec((1, D_MODEL), lambda i: (0, 0)),
            pl.BlockSpec((N_KINDS, D_MODEL), lambda i: (0, 0)),
        ],
        out_specs=pl.BlockSpec((N_KINDS, TR, D_MODEL), lambda i: (0, i, 0)),
        out_shape=jax.ShapeDtypeStruct((N_KINDS, MAX_TIME, D_MODEL),
                                       jnp.float32),
    )(time_emb, view_emb, kind_emb)
    return ct3.reshape(N_KINDS * MAX_TIME, D_MODEL)


def _gather_kernel(n_tok: int):
    tpw = n_tok // NW
    nch = tpw // CH

    mesh = plsc.VectorSubcoreMesh(core_axis_name="c", subcore_axis_name="s")

    scratch = [
        pltpu.VMEM((nch, CH), jnp.int32),
        pltpu.VMEM((nch, CH), jnp.int32),
    ]
    scratch += [pltpu.VMEM((CH, D_MODEL), jnp.float32) for _ in range(NB)]
    scratch += [pltpu.SemaphoreType.DMA for _ in range(2 * NB)]

    @functools.partial(
        pl.kernel,
        mesh=mesh,
        out_type=jax.ShapeDtypeStruct((n_tok, D_MODEL), jnp.float32),
        scratch_types=scratch,
    )
    def k(t_hbm, kind_hbm, ct_hbm, out_hbm, idx_v, kind_v, *bufs_and_sems):
        bufs = bufs_and_sems[:NB]
        gsem = bufs_and_sems[NB:2 * NB]
        wsem = bufs_and_sems[2 * NB:]

        wid = lax.axis_index("s") * NC + lax.axis_index("c")
        base_row = wid * nch

        pltpu.sync_copy(t_hbm.at[pl.ds(base_row, nch)], idx_v)
        pltpu.sync_copy(kind_hbm.at[pl.ds(base_row, nch)], kind_v)

        zero = jnp.zeros((L,), jnp.int32)
        def fuse_row(r, _):
            for g in range(CH // L):
                sl = pl.ds(g * L, L)
                tt = jnp.minimum(jnp.maximum(idx_v[r, sl], zero),
                                 jnp.full((L,), MAX_TIME - 1, jnp.int32))
                kk = jnp.minimum(jnp.maximum(kind_v[r, sl], zero),
                                 jnp.full((L,), N_KINDS - 1, jnp.int32))
                idx_v[r, sl] = tt + kk * MAX_TIME
            return 0
        for r in range(NB - 1):
            fuse_row(r, 0)

        def gather(c, b):
            return pltpu.async_copy(ct_hbm.at[idx_v.at[c]], bufs[b], gsem[b])

        def wb_start(c, b):
            return pltpu.async_copy(
                bufs[b], out_hbm.at[pl.ds(wid * tpw + c * CH, CH)], wsem[b])

        def wb_wait(c, b):
            pltpu.make_async_copy(
                bufs[b], out_hbm.at[pl.ds(wid * tpw + c * CH, CH)],
                wsem[b]).wait()

        for c in range(NB - 1):
            gather(c, c)

        def process(c, b):
            pltpu.make_async_copy(
                ct_hbm.at[idx_v.at[c]], bufs[b], gsem[b]).wait()
            wb_start(c, b)

            b_next = (b + NB - 1) % NB

            @pl.when(c + NB - 1 < nch)
            def _():
                fuse_row(c + NB - 1, 0)

                @pl.when(c >= 1)
                def _():
                    wb_wait(c - 1, b_next)
                gather(c + NB - 1, b_next)

        def outer(co, _):
            for b in range(NB):
                process(co * NB + b, b)
            return 0

        lax.fori_loop(0, nch // NB, outer, 0, unroll=False)

        for i in range(NB):
            c = nch - NB + i
            wb_wait(c, c % NB)

    return k


def kernel(t, kind_id, view_id, time_emb, view_emb, kind_emb):
    del view_id
    b, s = t.shape
    n_tok = b * s
    t2 = t.reshape(n_tok // CH, CH).astype(jnp.int32)
    k2 = kind_id.reshape(n_tok // CH, CH).astype(jnp.int32)
    ct = _combine_table(time_emb, view_emb, kind_emb)
    out = _gather_kernel(n_tok)(t2, k2, ct)
    return out.reshape(b, s, D_MODEL)
